# baseline (speedup 1.0000x reference)
import jax
import jax.numpy as jnp
from jax.experimental import pallas as pl
from jax.experimental.pallas import tpu as pltpu

_BR = 256
_BC = 8192
_CH = 128


def _scan_kernel(x_ref, tri_ref, o_ref, carry_ref):
    j = pl.program_id(1)

    @pl.when(j == 0)
    def _():
        carry_ref[...] = jnp.zeros_like(carry_ref)

    tri = tri_ref[...]
    carry = carry_ref[:, 0:1]
    for k in range(_BC // _CH):
        xc = x_ref[:, k * _CH : (k + 1) * _CH]
        part = jax.lax.dot_general(
            xc,
            tri,
            dimension_numbers=(((1,), (0,)), ((), ())),
            precision=jax.lax.Precision.DEFAULT,
            preferred_element_type=jnp.float32,
        )
        outc = part + carry
        o_ref[:, k * _CH : (k + 1) * _CH] = outc
        carry = outc[:, _CH - 1 : _CH]
    carry_ref[...] = jnp.broadcast_to(carry, carry_ref.shape)


def kernel(x):
    m, n = x.shape
    tri = jnp.triu(jnp.ones((_CH, _CH), dtype=jnp.float32))
    return pl.pallas_call(
        _scan_kernel,
        grid=(m // _BR, n // _BC),
        in_specs=[
            pl.BlockSpec((_BR, _BC), lambda i, j: (i, j)),
            pl.BlockSpec((_CH, _CH), lambda i, j: (0, 0)),
        ],
        out_specs=pl.BlockSpec((_BR, _BC), lambda i, j: (i, j)),
        out_shape=jax.ShapeDtypeStruct((m, n), jnp.float32),
        scratch_shapes=[pltpu.VMEM((_BR, 128), jnp.float32)],
        compiler_params=pltpu.CompilerParams(
            dimension_semantics=("parallel", "arbitrary"),
        ),
    )(x, tri)


# speedup vs baseline: 1.0954x; 1.0954x over previous
import jax
import jax.numpy as jnp
from jax.experimental import pallas as pl
from jax.experimental.pallas import tpu as pltpu

_BR = 1024
_BC = 2048
_CH = 128


def _scan_kernel(x_ref, tri_ref, o_ref, carry_ref):
    j = pl.program_id(1)

    @pl.when(j == 0)
    def _():
        carry_ref[...] = jnp.zeros_like(carry_ref)

    tri = tri_ref[...]
    carry = carry_ref[:, 0:1]
    for k in range(_BC // _CH):
        xc = x_ref[:, k * _CH : (k + 1) * _CH]
        part = jax.lax.dot_general(
            xc,
            tri,
            dimension_numbers=(((1,), (0,)), ((), ())),
            precision=jax.lax.Precision.DEFAULT,
            preferred_element_type=jnp.float32,
        )
        outc = part + carry
        o_ref[:, k * _CH : (k + 1) * _CH] = outc
        carry = outc[:, _CH - 1 : _CH]
    carry_ref[...] = jnp.broadcast_to(carry, carry_ref.shape)


def kernel(x):
    m, n = x.shape
    tri = jnp.triu(jnp.ones((_CH, _CH), dtype=jnp.float32))
    return pl.pallas_call(
        _scan_kernel,
        grid=(m // _BR, n // _BC),
        in_specs=[
            pl.BlockSpec((_BR, _BC), lambda i, j: (i, j)),
            pl.BlockSpec((_CH, _CH), lambda i, j: (0, 0)),
        ],
        out_specs=pl.BlockSpec((_BR, _BC), lambda i, j: (i, j)),
        out_shape=jax.ShapeDtypeStruct((m, n), jnp.float32),
        scratch_shapes=[pltpu.VMEM((_BR, 128), jnp.float32)],
        compiler_params=pltpu.CompilerParams(
            dimension_semantics=("parallel", "arbitrary"),
        ),
    )(x, tri)


# speedup vs baseline: 1.0967x; 1.0012x over previous
import jax
import jax.numpy as jnp
from jax.experimental import pallas as pl
from jax.experimental.pallas import tpu as pltpu

_BR = 2048
_BC = 1024
_CH = 128


def _scan_kernel(x_ref, tri_ref, o_ref, carry_ref):
    j = pl.program_id(1)

    @pl.when(j == 0)
    def _():
        carry_ref[...] = jnp.zeros_like(carry_ref)

    tri = tri_ref[...]
    carry = carry_ref[:, 0:1]
    for k in range(_BC // _CH):
        xc = x_ref[:, k * _CH : (k + 1) * _CH]
        part = jax.lax.dot_general(
            xc,
            tri,
            dimension_numbers=(((1,), (0,)), ((), ())),
            precision=jax.lax.Precision.DEFAULT,
            preferred_element_type=jnp.float32,
        )
        outc = part + carry
        o_ref[:, k * _CH : (k + 1) * _CH] = outc
        carry = outc[:, _CH - 1 : _CH]
    carry_ref[...] = jnp.broadcast_to(carry, carry_ref.shape)


def kernel(x):
    m, n = x.shape
    tri = jnp.triu(jnp.ones((_CH, _CH), dtype=jnp.float32))
    return pl.pallas_call(
        _scan_kernel,
        grid=(m // _BR, n // _BC),
        in_specs=[
            pl.BlockSpec((_BR, _BC), lambda i, j: (i, j)),
            pl.BlockSpec((_CH, _CH), lambda i, j: (0, 0)),
        ],
        out_specs=pl.BlockSpec((_BR, _BC), lambda i, j: (i, j)),
        out_shape=jax.ShapeDtypeStruct((m, n), jnp.float32),
        scratch_shapes=[pltpu.VMEM((_BR, 128), jnp.float32)],
        compiler_params=pltpu.CompilerParams(
            dimension_semantics=("parallel", "arbitrary"),
        ),
    )(x, tri)
